# Initial kernel scaffold; baseline (speedup 1.0000x reference)
import jax
import jax.numpy as jnp
from jax.experimental import pallas as pl


def kernel(x, edge_index, batch, W1l, b1l, W1r, W2l, b2l, W2r, W3l, b3l, W3r, Wg, bg, W4, b4, W5, b5):
    raise NotImplementedError("write your pallas kernel here")


# speedup vs baseline: 6.6088x; 6.6088x over previous
import functools

import jax
import jax.numpy as jnp
from jax import lax
from jax.experimental import pallas as pl
from jax.experimental.pallas import tpu as pltpu
from jax.experimental.pallas import tpu_sc as plsc

NC = 2
NS = 16
NW = NC * NS

CW = 16


def _pick_chunk(epw):
    for k in range(128, 0, -8):
        if epw % k == 0:
            return k
    raise ValueError(f"no valid chunk size for {epw} edges per worker")


@functools.lru_cache(maxsize=None)
def _make_sc_agg(n, e, h, gather=True):
    epw = e // NW
    k = _pick_chunk(epw)
    nchunk = epw // k
    nslab = n // k

    mesh = plsc.VectorSubcoreMesh(
        core_axis_name="c", subcore_axis_name="s", num_cores=NC, num_subcores=NS
    )

    out_type = jax.ShapeDtypeStruct((NC, n, h), jnp.float32)
    scratch = [
        pltpu.VMEM((nchunk, k), jnp.int32),
        pltpu.VMEM((k, h), jnp.float32),
        pltpu.VMEM_SHARED((n, h), jnp.float32),
        pltpu.SemaphoreType.DMA,
    ]
    if gather:
        scratch.insert(0, pltpu.VMEM((nchunk, k), jnp.int32))

    def body(*refs):
        if gather:
            (hs_hbm, src_hbm, dst_hbm, z_hbm, agg_out,
             sidx, didx, rows, acc_sh, sem) = refs
        else:
            (dst_hbm, z_hbm, ones_hbm, agg_out,
             didx, rows, acc_sh, sem) = refs

        c = lax.axis_index("c")
        s = lax.axis_index("s")
        wid = s * NC + c

        if gather:
            pltpu.sync_copy(src_hbm.at[wid], sidx)
        pltpu.sync_copy(dst_hbm.at[wid], didx)
        pltpu.sync_copy(z_hbm, rows)

        nz = (nslab - s + NS - 1) // NS

        def zslab(m, carry):
            sl = (s + m * NS) * k
            pltpu.sync_copy(rows, acc_sh.at[pl.ds(sl, k)])
            return carry

        lax.fori_loop(0, nz, zslab, 0)
        if not gather:
            pltpu.sync_copy(ones_hbm, rows)
        plsc.subcore_barrier()

        if gather:
            def chunk(j, carry):
                pltpu.async_copy(hs_hbm.at[sidx.at[j]], rows, sem).wait()
                pltpu.sync_copy(rows, acc_sh.at[didx.at[j]], add=True)
                return carry
        else:
            def chunk(j, carry):
                pltpu.sync_copy(rows, acc_sh.at[didx.at[j]], add=True)
                return carry

        lax.fori_loop(0, nchunk, chunk, 0)
        plsc.subcore_barrier()

        def wslab(m, carry):
            sl = (s + m * NS) * k
            pltpu.sync_copy(acc_sh.at[pl.ds(sl, k)], rows)
            pltpu.sync_copy(rows, agg_out.at[c, pl.ds(sl, k)])
            return carry

        lax.fori_loop(0, nz, wslab, 0)

    return pl.kernel(body, out_type=out_type, mesh=mesh,
                     scratch_types=tuple(scratch))


def _cntred_body(cnt_ref, inv_ref):
    s = cnt_ref[0, :, 0:1] + cnt_ref[1, :, 0:1]
    inv_ref[...] = 1.0 / jnp.maximum(s, 1.0)


@functools.lru_cache(maxsize=None)
def _make_cntred(n, h):
    nblk = n // BLK
    return pl.pallas_call(
        _cntred_body, grid=(nblk,),
        in_specs=[pl.BlockSpec((NC, BLK, h), lambda i: (0, i, 0))],
        out_specs=pl.BlockSpec((BLK, 1), lambda i: (i, 0)),
        out_shape=jax.ShapeDtypeStruct((n, 1), jnp.float32))


BLK = 1000


def _dense_body(p_ref, inv_ref, h_ref, wl_ref, bl_ref, wr_ref, out_ref):
    mean = (p_ref[0] + p_ref[1]) * inv_ref[...]
    acc = jnp.dot(mean, wl_ref[...], preferred_element_type=jnp.float32)
    acc += jnp.dot(h_ref[...], wr_ref[...], preferred_element_type=jnp.float32)
    out_ref[...] = jnp.maximum(acc + bl_ref[...], 0.0)


@functools.lru_cache(maxsize=None)
def _make_dense(n, h):
    nblk = n // BLK
    w_spec = pl.BlockSpec((h, h), lambda i: (0, 0))
    b_spec = pl.BlockSpec((1, h), lambda i: (0, 0))
    p_spec = pl.BlockSpec((NC, BLK, h), lambda i: (0, i, 0))
    h_spec = pl.BlockSpec((BLK, h), lambda i: (i, 0))
    col_spec = pl.BlockSpec((BLK, 1), lambda i: (i, 0))
    in_specs = [p_spec, col_spec, h_spec, w_spec, b_spec, w_spec]
    return pl.pallas_call(_dense_body, grid=(nblk,), in_specs=in_specs,
                          out_specs=h_spec,
                          out_shape=jax.ShapeDtypeStruct((n, h), jnp.float32))


def _final_body(p_ref, inv_ref, h_ref, wl_ref, bl_ref, wr_ref,
                wg_ref, bg_ref, w4_ref, b4_ref, w5_ref, b5_ref,
                batch_ref, out_ref, gmax_s, den_s, st_s):
    i = pl.program_id(0)
    nblk = pl.num_programs(0)
    g = den_s.shape[1]

    mean = (p_ref[0] + p_ref[1]) * inv_ref[...]
    acc = jnp.dot(mean, wl_ref[...], preferred_element_type=jnp.float32)
    acc += jnp.dot(h_ref[...], wr_ref[...], preferred_element_type=jnp.float32)
    h3 = jnp.maximum(acc + bl_ref[...], 0.0)

    gate = jnp.dot(h3, wg_ref[...], preferred_element_type=jnp.float32)
    gate += bg_ref[...]

    gid = lax.broadcasted_iota(jnp.int32, (h3.shape[0], g), 1)
    mask = batch_ref[...] == gid
    neg = jnp.float32(-jnp.inf)

    old_m = jnp.where(i == 0, neg, gmax_s[...])
    bm = jnp.max(jnp.where(mask, gate, neg), axis=0, keepdims=True)
    new_m = jnp.maximum(old_m, bm)
    scale = jnp.where(jnp.isfinite(new_m), jnp.exp(old_m - new_m), 0.0)

    gpn = jnp.max(jnp.where(mask, new_m, neg), axis=1, keepdims=True)
    e = jnp.exp(gate - gpn)
    we = jnp.where(mask, e, 0.0)

    old_d = jnp.where(i == 0, 0.0, den_s[...])
    old_st = jnp.where(i == 0, 0.0, st_s[...])
    den_s[...] = old_d * scale + jnp.sum(we, axis=0, keepdims=True)
    st_s[...] = old_st * scale + lax.dot_general(
        h3, we, (((0,), (0,)), ((), ())),
        preferred_element_type=jnp.float32)
    gmax_s[...] = new_m

    @pl.when(i == nblk - 1)
    def _():
        pooled_t = st_s[...] / jnp.maximum(den_s[...], 1e-16)
        r4 = lax.dot_general(pooled_t, w4_ref[...], (((0,), (0,)), ((), ())),
                             preferred_element_type=jnp.float32)
        r4 = jnp.maximum(r4 + b4_ref[...], 0.0)
        logits = jnp.dot(r4, w5_ref[...], preferred_element_type=jnp.float32)
        logits += b5_ref[...]
        m = jnp.max(logits, axis=-1, keepdims=True)
        z = logits - m
        lse = jnp.log(jnp.sum(jnp.exp(z), axis=-1, keepdims=True))
        out_ref[...] = z - lse


@functools.lru_cache(maxsize=None)
def _make_final(n, h, g, c):
    nblk = n // BLK
    w_spec = pl.BlockSpec((h, h), lambda i: (0, 0))
    b_spec = pl.BlockSpec((1, h), lambda i: (0, 0))
    in_specs = [
        pl.BlockSpec((NC, BLK, h), lambda i: (0, i, 0)),
        pl.BlockSpec((BLK, 1), lambda i: (i, 0)),
        pl.BlockSpec((BLK, h), lambda i: (i, 0)),
        w_spec, b_spec, w_spec,
        pl.BlockSpec((h, 1), lambda i: (0, 0)),
        pl.BlockSpec((1, 1), lambda i: (0, 0)),
        w_spec, b_spec,
        pl.BlockSpec((h, c), lambda i: (0, 0)),
        pl.BlockSpec((1, c), lambda i: (0, 0)),
        pl.BlockSpec((BLK, 1), lambda i: (i, 0)),
    ]
    return pl.pallas_call(
        _final_body, grid=(nblk,), in_specs=in_specs,
        out_specs=pl.BlockSpec((g, c), lambda i: (0, 0)),
        out_shape=jax.ShapeDtypeStruct((g, c), jnp.float32),
        scratch_shapes=[
            pltpu.VMEM((1, g), jnp.float32),
            pltpu.VMEM((1, g), jnp.float32),
            pltpu.VMEM((h, g), jnp.float32),
        ])


def kernel(x, edge_index, batch, W1l, b1l, W1r, W2l, b2l, W2r, W3l, b3l, W3r,
           Wg, bg, W4, b4, W5, b5):
    n, h = x.shape
    e = edge_index.shape[1]
    g = 64
    c = W5.shape[1]

    epw = e // NW
    k = _pick_chunk(epw)
    nchunk = epw // k

    src = edge_index[0].reshape(NW, nchunk, k)
    dst = edge_index[1].reshape(NW, nchunk, k)
    z = jnp.zeros((k, h), jnp.float32)
    ones = jnp.ones((k, h), jnp.float32)

    degree = _make_sc_agg(n, e, h, gather=False)
    cntred = _make_cntred(n, h)
    agg = _make_sc_agg(n, e, h)
    dense = _make_dense(n, h)
    final = _make_final(n, h, g, c)

    b1 = b1l.reshape(1, h)
    b2 = b2l.reshape(1, h)
    b3 = b3l.reshape(1, h)
    bg2 = bg.reshape(1, 1)
    b42 = b4.reshape(1, h)
    b52 = b5.reshape(1, c)
    bcol = batch.reshape(n, 1)

    cnt = degree(dst, z, ones)
    inv = cntred(cnt)
    p1 = agg(x, src, dst, z)
    h1 = dense(p1, inv, x, W1l, b1, W1r)
    p2 = agg(h1, src, dst, z)
    h2 = dense(p2, inv, h1, W2l, b2, W2r)
    p3 = agg(h2, src, dst, z)
    return final(p3, inv, h2, W3l, b3, W3r, Wg, bg2, W4, b42, W5, b52, bcol)


# speedup vs baseline: 8.2153x; 1.2431x over previous
import functools

import jax
import jax.numpy as jnp
from jax import lax
from jax.experimental import pallas as pl
from jax.experimental.pallas import tpu as pltpu
from jax.experimental.pallas import tpu_sc as plsc

NC = 2
NS = 16
NW = NC * NS

CW = 16


def _pick_chunk(epw, kmax=128):
    for k in range(kmax, 0, -8):
        if epw % k == 0:
            return k
    raise ValueError(f"no valid chunk size for {epw} edges per worker")


@functools.lru_cache(maxsize=None)
def _make_sc_agg(n, e, h, gather=True):
    epw = e // NW
    k = _pick_chunk(epw, 128)
    nchunk = epw // k
    nslab = n // k

    mesh = plsc.VectorSubcoreMesh(
        core_axis_name="c", subcore_axis_name="s", num_cores=NC, num_subcores=NS
    )

    out_type = jax.ShapeDtypeStruct((NC, n, h), jnp.float32)
    scratch = [
        pltpu.VMEM((nchunk, k), jnp.int32),
        pltpu.VMEM((k, h), jnp.float32),
        pltpu.VMEM_SHARED((n, h), jnp.float32),
        pltpu.SemaphoreType.DMA,
    ]
    if gather:
        scratch.insert(0, pltpu.VMEM((epw,), jnp.int32))
        scratch += [
            pltpu.VMEM((k, h), jnp.float32),
            pltpu.SemaphoreType.DMA,
        ]

    def body(*refs):
        if gather:
            (hs_hbm, src_hbm, dst_hbm, z_hbm, agg_out,
             sidx, didx, rows, acc_sh, sem, rows1, sem1) = refs
        else:
            (dst_hbm, z_hbm, ones_hbm, agg_out,
             didx, rows, acc_sh, sem) = refs

        c = lax.axis_index("c")
        s = lax.axis_index("s")
        wid = s * NC + c

        if gather:
            pltpu.sync_copy(src_hbm.at[pl.ds(wid * epw, epw)], sidx)
        pltpu.sync_copy(dst_hbm.at[wid], didx)
        pltpu.sync_copy(z_hbm, rows)

        nz = (nslab - s + NS - 1) // NS

        def zslab(m, carry):
            sl = (s + m * NS) * k
            pltpu.sync_copy(rows, acc_sh.at[pl.ds(sl, k)])
            return carry

        lax.fori_loop(0, nz, zslab, 0)
        if not gather:
            pltpu.sync_copy(ones_hbm, rows)
        if gather:
            pltpu.async_copy(hs_hbm.at[sidx.at[pl.ds(0, k)]], rows, sem)
        plsc.subcore_barrier()

        if gather:
            def gidx(j):
                return sidx.at[pl.ds(j * k, k)]

            def pair(m, carry):
                j0 = 2 * m
                pltpu.make_async_copy(hs_hbm.at[gidx(j0)], rows, sem).wait()
                pltpu.async_copy(hs_hbm.at[gidx(j0 + 1)], rows1, sem1)
                pltpu.sync_copy(rows, acc_sh.at[didx.at[j0]], add=True)
                pltpu.make_async_copy(
                    hs_hbm.at[gidx(j0 + 1)], rows1, sem1).wait()

                @pl.when(j0 + 2 < nchunk)
                def _():
                    pltpu.async_copy(hs_hbm.at[gidx(j0 + 2)], rows, sem)

                pltpu.sync_copy(rows1, acc_sh.at[didx.at[j0 + 1]], add=True)
                return carry

            lax.fori_loop(0, nchunk // 2, pair, 0)
            if nchunk % 2 == 1:
                j = nchunk - 1
                pltpu.make_async_copy(hs_hbm.at[gidx(j)], rows, sem).wait()
                pltpu.sync_copy(rows, acc_sh.at[didx.at[j]], add=True)
        else:
            def chunk(j, carry):
                pltpu.sync_copy(rows, acc_sh.at[didx.at[j]], add=True)
                return carry

            lax.fori_loop(0, nchunk, chunk, 0)
        plsc.subcore_barrier()

        def wslab(m, carry):
            sl = (s + m * NS) * k
            pltpu.sync_copy(acc_sh.at[pl.ds(sl, k)], rows)
            pltpu.sync_copy(rows, agg_out.at[c, pl.ds(sl, k)])
            return carry

        lax.fori_loop(0, nz, wslab, 0)

    return pl.kernel(body, out_type=out_type, mesh=mesh,
                     scratch_types=tuple(scratch))


def _cntred_body(cnt_ref, inv_ref):
    s = cnt_ref[0, :, 0:1] + cnt_ref[1, :, 0:1]
    inv_ref[...] = 1.0 / jnp.maximum(s, 1.0)


@functools.lru_cache(maxsize=None)
def _make_cntred(n, h):
    nblk = n // BLK
    return pl.pallas_call(
        _cntred_body, grid=(nblk,),
        in_specs=[pl.BlockSpec((NC, BLK, h), lambda i: (0, i, 0))],
        out_specs=pl.BlockSpec((BLK, 1), lambda i: (i, 0)),
        out_shape=jax.ShapeDtypeStruct((n, 1), jnp.float32))


BLK = 1000


def _dense_body(p_ref, inv_ref, h_ref, wl_ref, bl_ref, wr_ref, out_ref):
    mean = (p_ref[0] + p_ref[1]) * inv_ref[...]
    acc = jnp.dot(mean, wl_ref[...], preferred_element_type=jnp.float32)
    acc += jnp.dot(h_ref[...], wr_ref[...], preferred_element_type=jnp.float32)
    out_ref[...] = jnp.maximum(acc + bl_ref[...], 0.0)


@functools.lru_cache(maxsize=None)
def _make_dense(n, h):
    nblk = n // BLK
    w_spec = pl.BlockSpec((h, h), lambda i: (0, 0))
    b_spec = pl.BlockSpec((1, h), lambda i: (0, 0))
    p_spec = pl.BlockSpec((NC, BLK, h), lambda i: (0, i, 0))
    h_spec = pl.BlockSpec((BLK, h), lambda i: (i, 0))
    col_spec = pl.BlockSpec((BLK, 1), lambda i: (i, 0))
    in_specs = [p_spec, col_spec, h_spec, w_spec, b_spec, w_spec]
    return pl.pallas_call(_dense_body, grid=(nblk,), in_specs=in_specs,
                          out_specs=h_spec,
                          out_shape=jax.ShapeDtypeStruct((n, h), jnp.float32))


def _final_body(p_ref, inv_ref, h_ref, wl_ref, bl_ref, wr_ref,
                wg_ref, bg_ref, w4_ref, b4_ref, w5_ref, b5_ref,
                batch_ref, out_ref, gmax_s, den_s, st_s):
    i = pl.program_id(0)
    nblk = pl.num_programs(0)
    g = den_s.shape[1]

    mean = (p_ref[0] + p_ref[1]) * inv_ref[...]
    acc = jnp.dot(mean, wl_ref[...], preferred_element_type=jnp.float32)
    acc += jnp.dot(h_ref[...], wr_ref[...], preferred_element_type=jnp.float32)
    h3 = jnp.maximum(acc + bl_ref[...], 0.0)

    gate = jnp.dot(h3, wg_ref[...], preferred_element_type=jnp.float32)
    gate += bg_ref[...]

    gid = lax.broadcasted_iota(jnp.int32, (h3.shape[0], g), 1)
    mask = batch_ref[...] == gid
    neg = jnp.float32(-jnp.inf)

    old_m = jnp.where(i == 0, neg, gmax_s[...])
    bm = jnp.max(jnp.where(mask, gate, neg), axis=0, keepdims=True)
    new_m = jnp.maximum(old_m, bm)
    scale = jnp.where(jnp.isfinite(new_m), jnp.exp(old_m - new_m), 0.0)

    gpn = jnp.max(jnp.where(mask, new_m, neg), axis=1, keepdims=True)
    e = jnp.exp(gate - gpn)
    we = jnp.where(mask, e, 0.0)

    old_d = jnp.where(i == 0, 0.0, den_s[...])
    old_st = jnp.where(i == 0, 0.0, st_s[...])
    den_s[...] = old_d * scale + jnp.sum(we, axis=0, keepdims=True)
    st_s[...] = old_st * scale + lax.dot_general(
        h3, we, (((0,), (0,)), ((), ())),
        preferred_element_type=jnp.float32)
    gmax_s[...] = new_m

    @pl.when(i == nblk - 1)
    def _():
        pooled_t = st_s[...] / jnp.maximum(den_s[...], 1e-16)
        r4 = lax.dot_general(pooled_t, w4_ref[...], (((0,), (0,)), ((), ())),
                             preferred_element_type=jnp.float32)
        r4 = jnp.maximum(r4 + b4_ref[...], 0.0)
        logits = jnp.dot(r4, w5_ref[...], preferred_element_type=jnp.float32)
        logits += b5_ref[...]
        m = jnp.max(logits, axis=-1, keepdims=True)
        z = logits - m
        lse = jnp.log(jnp.sum(jnp.exp(z), axis=-1, keepdims=True))
        out_ref[...] = z - lse


@functools.lru_cache(maxsize=None)
def _make_final(n, h, g, c):
    nblk = n // BLK
    w_spec = pl.BlockSpec((h, h), lambda i: (0, 0))
    b_spec = pl.BlockSpec((1, h), lambda i: (0, 0))
    in_specs = [
        pl.BlockSpec((NC, BLK, h), lambda i: (0, i, 0)),
        pl.BlockSpec((BLK, 1), lambda i: (i, 0)),
        pl.BlockSpec((BLK, h), lambda i: (i, 0)),
        w_spec, b_spec, w_spec,
        pl.BlockSpec((h, 1), lambda i: (0, 0)),
        pl.BlockSpec((1, 1), lambda i: (0, 0)),
        w_spec, b_spec,
        pl.BlockSpec((h, c), lambda i: (0, 0)),
        pl.BlockSpec((1, c), lambda i: (0, 0)),
        pl.BlockSpec((BLK, 1), lambda i: (i, 0)),
    ]
    return pl.pallas_call(
        _final_body, grid=(nblk,), in_specs=in_specs,
        out_specs=pl.BlockSpec((g, c), lambda i: (0, 0)),
        out_shape=jax.ShapeDtypeStruct((g, c), jnp.float32),
        scratch_shapes=[
            pltpu.VMEM((1, g), jnp.float32),
            pltpu.VMEM((1, g), jnp.float32),
            pltpu.VMEM((h, g), jnp.float32),
        ])


def kernel(x, edge_index, batch, W1l, b1l, W1r, W2l, b2l, W2r, W3l, b3l, W3r,
           Wg, bg, W4, b4, W5, b5):
    n, h = x.shape
    e = edge_index.shape[1]
    g = 64
    c = W5.shape[1]

    epw = e // NW
    k = _pick_chunk(epw, 128)

    src = edge_index[0]
    dst = edge_index[1].reshape(NW, epw // k, k)
    z = jnp.zeros((k, h), jnp.float32)
    ones = jnp.ones((k, h), jnp.float32)

    degree = _make_sc_agg(n, e, h, gather=False)
    cntred = _make_cntred(n, h)
    agg = _make_sc_agg(n, e, h)
    dense = _make_dense(n, h)
    final = _make_final(n, h, g, c)

    b1 = b1l.reshape(1, h)
    b2 = b2l.reshape(1, h)
    b3 = b3l.reshape(1, h)
    bg2 = bg.reshape(1, 1)
    b42 = b4.reshape(1, h)
    b52 = b5.reshape(1, c)
    bcol = batch.reshape(n, 1)

    cnt = degree(dst, z, ones)
    inv = cntred(cnt)
    p1 = agg(x, src, dst, z)
    h1 = dense(p1, inv, x, W1l, b1, W1r)
    p2 = agg(h1, src, dst, z)
    h2 = dense(p2, inv, h1, W2l, b2, W2r)
    p3 = agg(h2, src, dst, z)
    return final(p3, inv, h2, W3l, b3, W3r, Wg, bg2, W4, b42, W5, b52, bcol)
